# Initial kernel scaffold; baseline (speedup 1.0000x reference)
import jax
import jax.numpy as jnp
from jax.experimental import pallas as pl


def kernel(x, Wx, Wh, b):
    raise NotImplementedError("write your pallas kernel here")


# speedup vs baseline: 4.1231x; 4.1231x over previous
import functools

import jax
import jax.numpy as jnp
from jax.experimental import pallas as pl
from jax.experimental.pallas import tpu as pltpu

_T = 8
_NB = 2


def _lstm_body(x_ref, wx_ref, wh_ref, b_ref, h_out, c_out,
               xp_ref, h_ref, c_ref, *, T, Bb, U):
    it = pl.program_id(1)
    nt = pl.num_programs(1)

    @pl.when(it == 0)
    def _init():
        h_ref[...] = jnp.zeros_like(h_ref)
        c_ref[...] = jnp.zeros_like(c_ref)

    xs = x_ref[...].reshape(T * Bb, x_ref.shape[2])
    xp_ref[...] = (
        jnp.dot(xs, wx_ref[...], preferred_element_type=jnp.float32)
        + b_ref[...]
    )

    for t in range(T):
        h_b = h_ref[...].astype(jnp.bfloat16)
        gates = xp_ref[pl.ds(t * Bb, Bb), :] + jnp.dot(
            h_b, wh_ref[...], preferred_element_type=jnp.float32)
        i = jax.nn.sigmoid(gates[:, :U])
        f = jax.nn.sigmoid(gates[:, U:2 * U])
        g = jnp.tanh(gates[:, 2 * U:3 * U])
        o = jax.nn.sigmoid(gates[:, 3 * U:])
        c_new = f * c_ref[...] + i * g
        c_ref[...] = c_new
        h_ref[...] = o * jnp.tanh(c_new)

    @pl.when(it == nt - 1)
    def _write():
        h_out[...] = h_ref[...]
        c_out[...] = c_ref[...]


@jax.jit
def kernel(x, Wx, Wh, b):
    B, S, D = x.shape
    U = Wh.shape[0]
    G = 4 * U
    T = _T
    NB = _NB
    Bb = B // NB

    xT = jnp.swapaxes(x, 0, 1).astype(jnp.bfloat16)
    wx = Wx.astype(jnp.bfloat16)
    wh = Wh.astype(jnp.bfloat16)
    b2 = b.astype(jnp.float32).reshape(1, G)

    body = functools.partial(_lstm_body, T=T, Bb=Bb, U=U)
    h, c = pl.pallas_call(
        body,
        out_shape=[
            jax.ShapeDtypeStruct((B, U), jnp.float32),
            jax.ShapeDtypeStruct((B, U), jnp.float32),
        ],
        grid=(NB, S // T),
        in_specs=[
            pl.BlockSpec((T, Bb, D), lambda ib, it: (it, ib, 0)),
            pl.BlockSpec((D, G), lambda ib, it: (0, 0)),
            pl.BlockSpec((U, G), lambda ib, it: (0, 0)),
            pl.BlockSpec((1, G), lambda ib, it: (0, 0)),
        ],
        out_specs=[
            pl.BlockSpec((Bb, U), lambda ib, it: (ib, 0)),
            pl.BlockSpec((Bb, U), lambda ib, it: (ib, 0)),
        ],
        scratch_shapes=[
            pltpu.VMEM((T * Bb, G), jnp.float32),
            pltpu.VMEM((Bb, U), jnp.float32),
            pltpu.VMEM((Bb, U), jnp.float32),
        ],
        compiler_params=pltpu.CompilerParams(
            dimension_semantics=("parallel", "arbitrary"),
            vmem_limit_bytes=56 * 1024 * 1024,
        ),
        name="lstm_fused",
    )(xT, wx, wh, b2)
    return h, c


# speedup vs baseline: 7.5630x; 1.8343x over previous
import functools

import jax
import jax.numpy as jnp
from jax.experimental import pallas as pl
from jax.experimental.pallas import tpu as pltpu

_T = 8


def _sigmoid(x):
    return 0.5 * jnp.tanh(0.5 * x) + 0.5


def _lstm_body(x_ref, wx_ref, wh_ref, b_ref, h_out, c_out,
               xp_ref, h_ref, c_ref, *, T, B, U):
    it = pl.program_id(0)
    nt = pl.num_programs(0)

    @pl.when(it == 0)
    def _init():
        h_ref[...] = jnp.zeros_like(h_ref)
        c_ref[...] = jnp.zeros_like(c_ref)

    xs = x_ref[...].reshape(T * B, x_ref.shape[2])
    xp_ref[...] = (
        jnp.dot(xs, wx_ref[...], preferred_element_type=jnp.float32)
        + b_ref[...]
    )

    for t in range(T):
        h_b = h_ref[...].astype(jnp.bfloat16)
        gates = xp_ref[pl.ds(t * B, B), :] + jnp.dot(
            h_b, wh_ref[...], preferred_element_type=jnp.float32)
        i = _sigmoid(gates[:, :U])
        f = _sigmoid(gates[:, U:2 * U])
        g = jnp.tanh(gates[:, 2 * U:3 * U])
        o = _sigmoid(gates[:, 3 * U:])
        c_new = f * c_ref[...] + i * g
        c_ref[...] = c_new
        h_ref[...] = o * jnp.tanh(c_new)

    @pl.when(it == nt - 1)
    def _write():
        h_out[...] = h_ref[...]
        c_out[...] = c_ref[...]


@jax.jit
def kernel(x, Wx, Wh, b):
    B, S, D = x.shape
    U = Wh.shape[0]
    G = 4 * U
    T = _T

    xT = jnp.swapaxes(x, 0, 1).astype(jnp.bfloat16)
    wx = Wx.astype(jnp.bfloat16)
    wh = Wh.astype(jnp.bfloat16)
    b2 = b.astype(jnp.float32).reshape(1, G)

    body = functools.partial(_lstm_body, T=T, B=B, U=U)
    h, c = pl.pallas_call(
        body,
        out_shape=[
            jax.ShapeDtypeStruct((B, U), jnp.float32),
            jax.ShapeDtypeStruct((B, U), jnp.float32),
        ],
        grid=(S // T,),
        in_specs=[
            pl.BlockSpec((T, B, D), lambda it: (it, 0, 0)),
            pl.BlockSpec((D, G), lambda it: (0, 0)),
            pl.BlockSpec((U, G), lambda it: (0, 0)),
            pl.BlockSpec((1, G), lambda it: (0, 0)),
        ],
        out_specs=[
            pl.BlockSpec((B, U), lambda it: (0, 0)),
            pl.BlockSpec((B, U), lambda it: (0, 0)),
        ],
        scratch_shapes=[
            pltpu.VMEM((T * B, G), jnp.float32),
            pltpu.VMEM((B, U), jnp.float32),
            pltpu.VMEM((B, U), jnp.float32),
        ],
        compiler_params=pltpu.CompilerParams(
            dimension_semantics=("arbitrary",),
            vmem_limit_bytes=56 * 1024 * 1024,
        ),
        name="lstm_fused",
    )(xT, wx, wh, b2)
    return h, c


# speedup vs baseline: 7.8902x; 1.0433x over previous
import functools

import jax
import jax.numpy as jnp
from jax.experimental import pallas as pl
from jax.experimental.pallas import tpu as pltpu

_T = 8


def _sigmoid(x):
    return 0.5 * jnp.tanh(0.5 * x) + 0.5


def _lstm_body(x_ref, p_ref, wx_ref, wh_ref, b_ref, h_out, c_out,
               xp_ref, h_ref, c_ref, *, T, B, U):
    it = pl.program_id(0)
    nt = pl.num_programs(0)

    @pl.when(it == 0)
    def _init():
        h_ref[...] = jnp.zeros_like(h_ref)
        c_ref[...] = jnp.zeros_like(c_ref)

    xs_b = x_ref[...].astype(jnp.bfloat16).reshape(B * T, x_ref.shape[2])
    xs_t = jnp.dot(p_ref[...], xs_b,
                   preferred_element_type=jnp.float32).astype(jnp.bfloat16)
    xp_ref[...] = (
        jnp.dot(xs_t, wx_ref[...], preferred_element_type=jnp.float32)
        + b_ref[...]
    )

    for t in range(T):
        h_b = h_ref[...].astype(jnp.bfloat16)
        gates = xp_ref[pl.ds(t * B, B), :] + jnp.dot(
            h_b, wh_ref[...], preferred_element_type=jnp.float32)
        i = _sigmoid(gates[:, :U])
        f = _sigmoid(gates[:, U:2 * U])
        g = jnp.tanh(gates[:, 2 * U:3 * U])
        o = _sigmoid(gates[:, 3 * U:])
        c_new = f * c_ref[...] + i * g
        c_ref[...] = c_new
        h_ref[...] = o * jnp.tanh(c_new)

    @pl.when(it == nt - 1)
    def _write():
        h_out[...] = h_ref[...]
        c_out[...] = c_ref[...]


@jax.jit
def kernel(x, Wx, Wh, b):
    B, S, D = x.shape
    U = Wh.shape[0]
    G = 4 * U
    T = _T

    wx = Wx.astype(jnp.bfloat16)
    wh = Wh.astype(jnp.bfloat16)
    b2 = b.astype(jnp.float32).reshape(1, G)
    rows = jnp.arange(T * B)
    src = (rows % B) * T + rows // B
    perm = (src[:, None] == jnp.arange(B * T)[None, :]).astype(jnp.bfloat16)

    body = functools.partial(_lstm_body, T=T, B=B, U=U)
    h, c = pl.pallas_call(
        body,
        out_shape=[
            jax.ShapeDtypeStruct((B, U), jnp.float32),
            jax.ShapeDtypeStruct((B, U), jnp.float32),
        ],
        grid=(S // T,),
        in_specs=[
            pl.BlockSpec((B, T, D), lambda it: (0, it, 0)),
            pl.BlockSpec((T * B, T * B), lambda it: (0, 0)),
            pl.BlockSpec((D, G), lambda it: (0, 0)),
            pl.BlockSpec((U, G), lambda it: (0, 0)),
            pl.BlockSpec((1, G), lambda it: (0, 0)),
        ],
        out_specs=[
            pl.BlockSpec((B, U), lambda it: (0, 0)),
            pl.BlockSpec((B, U), lambda it: (0, 0)),
        ],
        scratch_shapes=[
            pltpu.VMEM((T * B, G), jnp.float32),
            pltpu.VMEM((B, U), jnp.float32),
            pltpu.VMEM((B, U), jnp.float32),
        ],
        compiler_params=pltpu.CompilerParams(
            dimension_semantics=("arbitrary",),
            vmem_limit_bytes=56 * 1024 * 1024,
        ),
        name="lstm_fused",
    )(x, perm, wx, wh, b2)
    return h, c
